# baseline (speedup 1.0000x reference)
import functools

import jax
import jax.numpy as jnp
from jax.experimental import pallas as pl


def _body(seg, H, C, Rc, x_ref, wm_ref, ww_ref, bm_ref, bw_ref, out_ref):
    Sc = Rc // seg
    r_idx = jax.lax.broadcasted_iota(jnp.int32, (Sc, Rc), 1)
    s_idx = jax.lax.broadcasted_iota(jnp.int32, (Sc, Rc), 0)
    M = (r_idx // seg == s_idx).astype(jnp.bfloat16)
    wc = jnp.concatenate([wm_ref[...], ww_ref[...]], axis=1)
    bc = jnp.concatenate([bm_ref[...], bw_ref[...]], axis=1)
    lane = jax.lax.broadcasted_iota(jnp.int32, (Rc, H + 1), 1)
    for c in range(C):
        x = x_ref[pl.ds(c * Rc, Rc), :]
        a = jnp.dot(x, wc, preferred_element_type=jnp.float32) + bc
        sig = jax.nn.sigmoid(a)
        act = jnp.where(lane < H, a * sig, 1.0)
        z = act * sig[:, H:H + 1]
        nd = jnp.dot(M, z.astype(jnp.bfloat16),
                     preferred_element_type=jnp.float32)
        out_ref[pl.ds(c * Sc, Sc), :] = nd[:, :H] / nd[:, H:H + 1]


def kernel(atoms, n_atoms, W_mlp, b_mlp, W_w, b_w):
    N, D = atoms.shape
    B = n_atoms.shape[0]
    H = W_mlp.shape[1]
    seg = N // B
    R = 20000
    Rc = 4000
    S = R // seg
    C = R // Rc

    bm = b_mlp[None, :]
    bw = b_w[None, :]

    body = functools.partial(_body, seg, H, C, Rc)
    out = pl.pallas_call(
        body,
        grid=(N // R,),
        in_specs=[
            pl.BlockSpec((R, D), lambda i: (i, 0)),
            pl.BlockSpec((D, H), lambda i: (0, 0)),
            pl.BlockSpec((D, 1), lambda i: (0, 0)),
            pl.BlockSpec((1, H), lambda i: (0, 0)),
            pl.BlockSpec((1, 1), lambda i: (0, 0)),
        ],
        out_specs=pl.BlockSpec((S, H), lambda i: (i, 0)),
        out_shape=jax.ShapeDtypeStruct((B, H), jnp.float32),
    )(atoms, W_mlp, W_w, bm, bw)
    return out


# speedup vs baseline: 1.2492x; 1.2492x over previous
import functools

import jax
import jax.numpy as jnp
from jax.experimental import pallas as pl


def _body(seg, H, C, Rc, x_ref, wc_ref, bc_ref, out_ref):
    Sc = Rc // seg
    r_idx = jax.lax.broadcasted_iota(jnp.int32, (Rc, Sc), 0)
    s_idx = jax.lax.broadcasted_iota(jnp.int32, (Rc, Sc), 1)
    MT = (r_idx // seg == s_idx).astype(jnp.bfloat16)
    wc = wc_ref[...]
    bc = bc_ref[...]
    subl = jax.lax.broadcasted_iota(jnp.int32, (H + 1, Rc), 0)
    for c in range(C):
        x = x_ref[pl.ds(c * Rc, Rc), :]
        aT = jax.lax.dot_general(wc, x, (((0,), (1,)), ((), ())),
                                 preferred_element_type=jnp.float32) + bc
        sigT = jax.nn.sigmoid(aT)
        actT = jnp.where(subl < H, aT * sigT, 1.0)
        zT = actT * sigT[H:H + 1, :]
        ndT = jax.lax.dot_general(zT.astype(jnp.bfloat16), MT,
                                  (((1,), (0,)), ((), ())),
                                  preferred_element_type=jnp.float32)
        out_ref[pl.ds(c * Sc, Sc), :] = (ndT[:H, :] / ndT[H:H + 1, :]).T


def kernel(atoms, n_atoms, W_mlp, b_mlp, W_w, b_w):
    N, D = atoms.shape
    B = n_atoms.shape[0]
    H = W_mlp.shape[1]
    seg = N // B
    R = 20000
    Rc = 4000
    S = R // seg
    C = R // Rc

    Wc = jnp.concatenate([W_mlp, W_w], axis=1)
    bc = jnp.concatenate([b_mlp, b_w])[:, None]

    body = functools.partial(_body, seg, H, C, Rc)
    out = pl.pallas_call(
        body,
        grid=(N // R,),
        in_specs=[
            pl.BlockSpec((R, D), lambda i: (i, 0)),
            pl.BlockSpec((D, H + 1), lambda i: (0, 0)),
            pl.BlockSpec((H + 1, 1), lambda i: (0, 0)),
        ],
        out_specs=pl.BlockSpec((S, H), lambda i: (i, 0)),
        out_shape=jax.ShapeDtypeStruct((B, H), jnp.float32),
    )(atoms, Wc, bc)
    return out


# speedup vs baseline: 1.2871x; 1.0303x over previous
import functools

import jax
import jax.numpy as jnp
from jax.experimental import pallas as pl


def _body(seg, H, C, Rc, x_ref, aux_ref, out_ref):
    Sc = Rc // seg
    r_idx = jax.lax.broadcasted_iota(jnp.int32, (Rc, Sc), 0)
    s_idx = jax.lax.broadcasted_iota(jnp.int32, (Rc, Sc), 1)
    MT = (r_idx // seg == s_idx).astype(jnp.bfloat16)
    D = aux_ref.shape[0] - 1
    wc = aux_ref[:D, :]
    bc = aux_ref[D:D + 1, :].T
    subl = jax.lax.broadcasted_iota(jnp.int32, (H + 1, Rc), 0)
    for c in range(C):
        x = x_ref[pl.ds(c * Rc, Rc), :]
        aT = jax.lax.dot_general(wc, x, (((0,), (1,)), ((), ())),
                                 preferred_element_type=jnp.float32) + bc
        sigT = jax.nn.sigmoid(aT)
        actT = jnp.where(subl < H, aT * sigT, 1.0)
        zT = actT * sigT[H:H + 1, :]
        ndT = jax.lax.dot_general(zT.astype(jnp.bfloat16), MT,
                                  (((1,), (0,)), ((), ())),
                                  preferred_element_type=jnp.float32)
        out_ref[pl.ds(c * Sc, Sc), :] = (ndT[:H, :] / ndT[H:H + 1, :]).T


def kernel(atoms, n_atoms, W_mlp, b_mlp, W_w, b_w):
    N, D = atoms.shape
    B = n_atoms.shape[0]
    H = W_mlp.shape[1]
    seg = N // B
    R = 20000
    Rc = 4000
    S = R // seg
    C = R // Rc

    aux = jnp.concatenate(
        [jnp.concatenate([W_mlp, W_w], axis=1),
         jnp.concatenate([b_mlp, b_w])[None, :]], axis=0)

    body = functools.partial(_body, seg, H, C, Rc)
    out = pl.pallas_call(
        body,
        grid=(N // R,),
        in_specs=[
            pl.BlockSpec((R, D), lambda i: (i, 0)),
            pl.BlockSpec((D + 1, H + 1), lambda i: (0, 0)),
        ],
        out_specs=pl.BlockSpec((S, H), lambda i: (i, 0)),
        out_shape=jax.ShapeDtypeStruct((B, H), jnp.float32),
    )(atoms, aux)
    return out


# speedup vs baseline: 1.2923x; 1.0041x over previous
import functools

import jax
import jax.numpy as jnp
from jax.experimental import pallas as pl


def _body(seg, H, C, Rc, x_ref, aux_ref, out_ref):
    Sc = Rc // seg
    r_idx = jax.lax.broadcasted_iota(jnp.int32, (Rc, Sc), 0)
    s_idx = jax.lax.broadcasted_iota(jnp.int32, (Rc, Sc), 1)
    MT = (r_idx // seg == s_idx).astype(jnp.bfloat16)
    D = aux_ref.shape[0] - 1
    wc = aux_ref[:D, :]
    bc = aux_ref[D:D + 1, :].T
    subl = jax.lax.broadcasted_iota(jnp.int32, (H + 1, Rc), 0)
    for c in range(C):
        x = x_ref[pl.ds(c * Rc, Rc), :]
        aT = jax.lax.dot_general(wc, x, (((0,), (1,)), ((), ())),
                                 preferred_element_type=jnp.float32) + bc
        sigT = jax.nn.sigmoid(aT)
        actT = jnp.where(subl < H, aT * sigT, 1.0)
        zT = actT * sigT[H:H + 1, :]
        ndT = jax.lax.dot_general(zT.astype(jnp.bfloat16), MT,
                                  (((1,), (0,)), ((), ())),
                                  preferred_element_type=jnp.float32)
        out_ref[0, pl.ds(c * Sc, Sc), :] = (ndT[:H, :] / ndT[H:H + 1, :]).T


def kernel(atoms, n_atoms, W_mlp, b_mlp, W_w, b_w):
    N, D = atoms.shape
    B = n_atoms.shape[0]
    H = W_mlp.shape[1]
    seg = N // B
    R = 25000
    Rc = 5000
    S = R // seg
    C = R // Rc

    aux = jnp.concatenate(
        [jnp.concatenate([W_mlp, W_w], axis=1),
         jnp.concatenate([b_mlp, b_w])[None, :]], axis=0)

    body = functools.partial(_body, seg, H, C, Rc)
    out = pl.pallas_call(
        body,
        grid=(N // R,),
        in_specs=[
            pl.BlockSpec((R, D), lambda i: (i, 0)),
            pl.BlockSpec((D + 1, H + 1), lambda i: (0, 0)),
        ],
        out_specs=pl.BlockSpec((1, S, H), lambda i: (i, 0, 0)),
        out_shape=jax.ShapeDtypeStruct((N // R, S, H), jnp.float32),
    )(atoms, aux)
    return out.reshape(B, H)


# speedup vs baseline: 1.4116x; 1.0923x over previous
import functools

import jax
import jax.numpy as jnp
from jax.experimental import pallas as pl


def _body(seg, H, C, Rc, x1_ref, x2_ref, aux_ref, out_ref):
    Sc = Rc // seg
    r_idx = jax.lax.broadcasted_iota(jnp.int32, (Rc, Sc), 0)
    s_idx = jax.lax.broadcasted_iota(jnp.int32, (Rc, Sc), 1)
    MT = (r_idx // seg == s_idx).astype(jnp.bfloat16)
    D = aux_ref.shape[0] - 1
    wc = aux_ref[:D, :]
    bc = aux_ref[D:D + 1, :].T
    subl = jax.lax.broadcasted_iota(jnp.int32, (H + 1, Rc), 0)
    for h, x_ref in enumerate((x1_ref, x2_ref)):
        for c in range(C):
            x = x_ref[pl.ds(c * Rc, Rc), :]
            aT = jax.lax.dot_general(wc, x, (((0,), (1,)), ((), ())),
                                     preferred_element_type=jnp.float32) + bc
            sigT = jax.nn.sigmoid(aT)
            actT = jnp.where(subl < H, aT * sigT, 1.0)
            zT = actT * sigT[H:H + 1, :]
            ndT = jax.lax.dot_general(zT.astype(jnp.bfloat16), MT,
                                      (((1,), (0,)), ((), ())),
                                      preferred_element_type=jnp.float32)
            out_ref[0, pl.ds((h * C + c) * Sc, Sc), :] = (
                ndT[:H, :] / ndT[H:H + 1, :]).T


def kernel(atoms, n_atoms, W_mlp, b_mlp, W_w, b_w):
    N, D = atoms.shape
    B = n_atoms.shape[0]
    H = W_mlp.shape[1]
    seg = N // B
    R = 20000
    Rh = R // 2
    Rc = 5000
    S = R // seg
    C = Rh // Rc

    aux = jnp.concatenate(
        [jnp.concatenate([W_mlp, W_w], axis=1),
         jnp.concatenate([b_mlp, b_w])[None, :]], axis=0)

    body = functools.partial(_body, seg, H, C, Rc)
    out = pl.pallas_call(
        body,
        grid=(N // R,),
        in_specs=[
            pl.BlockSpec((Rh, D), lambda i: (2 * i, 0)),
            pl.BlockSpec((Rh, D), lambda i: (2 * i + 1, 0)),
            pl.BlockSpec((D + 1, H + 1), lambda i: (0, 0)),
        ],
        out_specs=pl.BlockSpec((1, S, H), lambda i: (i, 0, 0)),
        out_shape=jax.ShapeDtypeStruct((N // R, S, H), jnp.float32),
    )(atoms, atoms, aux)
    return out.reshape(B, H)
